# Initial kernel scaffold; baseline (speedup 1.0000x reference)
import jax
import jax.numpy as jnp
from jax.experimental import pallas as pl


def kernel(x, adj_mat, Wq, bq, Wk, bk, Wv, bv, Wskip, bskip, ln_g, ln_b):
    raise NotImplementedError("write your pallas kernel here")


# speedup vs baseline: 2015.1364x; 2015.1364x over previous
import jax
import jax.numpy as jnp
from jax.experimental import pallas as pl

N = 512
DIM = 64
HEADS = 8


def _attn_kernel(x_ref, adjt_ref, wq_ref, bq_ref, wk_ref, bk_ref,
                 wv_ref, bv_ref, wskip_ref, bskip_ref, lng_ref, lnb_ref,
                 o_ref):
    x = x_ref[...]
    q = jnp.dot(x, wq_ref[...], preferred_element_type=jnp.float32) + bq_ref[...]
    k = jnp.dot(x, wk_ref[...], preferred_element_type=jnp.float32) + bk_ref[...]
    v = jnp.dot(x, wv_ref[...], preferred_element_type=jnp.float32) + bv_ref[...]
    mask = adjt_ref[...] != 0

    acc = jnp.zeros((N, DIM), dtype=jnp.float32)
    for h in range(HEADS):
        sl = slice(h * DIM, (h + 1) * DIM)
        qh, kh, vh = q[:, sl], k[:, sl], v[:, sl]
        s = jax.lax.dot_general(
            qh, kh, (((1,), (1,)), ((), ())),
            preferred_element_type=jnp.float32) * 0.125
        s_masked = jnp.where(mask, s, -jnp.inf)
        m = jnp.max(s_masked, axis=1, keepdims=True)
        m = jnp.where(jnp.isfinite(m), m, 0.0)
        ex = jnp.where(mask, jnp.exp(s - m), 0.0)
        den = jnp.sum(ex, axis=1, keepdims=True)
        alpha = ex / jnp.where(den > 0, den, 1.0)
        acc = acc + jnp.dot(alpha, vh, preferred_element_type=jnp.float32)

    out = acc * (1.0 / HEADS) \
        + jnp.dot(x, wskip_ref[...], preferred_element_type=jnp.float32) \
        + bskip_ref[...]
    mu = jnp.mean(out, axis=1, keepdims=True)
    c = out - mu
    var = jnp.mean(c * c, axis=1, keepdims=True)
    y = c * jax.lax.rsqrt(var + 1e-5) * lng_ref[...] + lnb_ref[...]
    o_ref[...] = y + x


def kernel(x, adj_mat, Wq, bq, Wk, bk, Wv, bv, Wskip, bskip, ln_g, ln_b):
    x0 = x[0]
    adjt = adj_mat[0].T
    y = pl.pallas_call(
        _attn_kernel,
        out_shape=jax.ShapeDtypeStruct((N, DIM), jnp.float32),
    )(x0, adjt,
      Wq, bq.reshape(1, HEADS * DIM),
      Wk, bk.reshape(1, HEADS * DIM),
      Wv, bv.reshape(1, HEADS * DIM),
      Wskip, bskip.reshape(1, DIM),
      ln_g.reshape(1, DIM), ln_b.reshape(1, DIM))
    return y[None]


# speedup vs baseline: 2828.2690x; 1.4035x over previous
import jax
import jax.numpy as jnp
from jax.experimental import pallas as pl

N = 512
DIM = 64
HEADS = 8

_NEG = -1e30


def _attn_kernel(x_ref, adj_ref, wq_ref, bq_ref, wk_ref, bk_ref,
                 wv_ref, bv_ref, wskip_ref, bskip_ref, lng_ref, lnb_ref,
                 o_ref):
    x = x_ref[...]
    q = jnp.dot(x, wq_ref[...], preferred_element_type=jnp.float32) + bq_ref[...]
    k = jnp.dot(x, wk_ref[...], preferred_element_type=jnp.float32) + bk_ref[...]
    v = jnp.dot(x, wv_ref[...], preferred_element_type=jnp.float32) + bv_ref[...]
    bias = jnp.where(adj_ref[...] != 0, 0.0, _NEG)

    acc = jnp.zeros((N, DIM), dtype=jnp.float32)
    for h in range(HEADS):
        sl = slice(h * DIM, (h + 1) * DIM)
        qh, kh, vh = q[:, sl], k[:, sl], v[:, sl]
        st = jax.lax.dot_general(
            kh, qh, (((1,), (1,)), ((), ())),
            preferred_element_type=jnp.float32) * 0.125 + bias
        m = jnp.max(st, axis=0, keepdims=True)
        m = jnp.where(m < -1e29, 0.0, m)
        ex = jnp.exp(st - m)
        den = jnp.sum(ex, axis=0, keepdims=True)
        recip = 1.0 / jnp.where(den > 0, den, 1.0)
        agg = jax.lax.dot_general(
            ex * recip, vh, (((0,), (0,)), ((), ())),
            preferred_element_type=jnp.float32)
        acc = acc + agg

    out = acc * (1.0 / HEADS) \
        + jnp.dot(x, wskip_ref[...], preferred_element_type=jnp.float32) \
        + bskip_ref[...]
    mu = jnp.mean(out, axis=1, keepdims=True)
    c = out - mu
    var = jnp.mean(c * c, axis=1, keepdims=True)
    y = c * jax.lax.rsqrt(var + 1e-5) * lng_ref[...] + lnb_ref[...]
    o_ref[...] = y + x


def kernel(x, adj_mat, Wq, bq, Wk, bk, Wv, bv, Wskip, bskip, ln_g, ln_b):
    y = pl.pallas_call(
        _attn_kernel,
        out_shape=jax.ShapeDtypeStruct((N, DIM), jnp.float32),
    )(x[0], adj_mat[0],
      Wq, bq.reshape(1, HEADS * DIM),
      Wk, bk.reshape(1, HEADS * DIM),
      Wv, bv.reshape(1, HEADS * DIM),
      Wskip, bskip.reshape(1, DIM),
      ln_g.reshape(1, DIM), ln_b.reshape(1, DIM))
    return y[None]
